# baseline (speedup 1.0000x reference)
import functools

import jax
import jax.numpy as jnp
from jax import lax
from jax.experimental import pallas as pl
from jax.experimental.pallas import tpu as pltpu
from jax.experimental.pallas import tpu_sc as plsc

B = 16384
D = 32
H1 = 64
H2 = 32

NC = 2
NS = 16
NW = NC * NS
BPW = B // NW
CHUNK = 128
NCH = BPW // CHUNK

BLK = 2048
GRID = B // BLK


def _build_gather():
  mesh = plsc.VectorSubcoreMesh(core_axis_name="c", subcore_axis_name="s")

  @functools.partial(
      pl.kernel,
      mesh=mesh,
      out_type=[
          jax.ShapeDtypeStruct((NW, BPW, D), jnp.float32),
          jax.ShapeDtypeStruct((NW, BPW, D), jnp.float32),
      ],
      scratch_types=[
          pltpu.VMEM((NCH, CHUNK), jnp.int32),
          pltpu.VMEM((NCH, CHUNK), jnp.int32),
          pltpu.VMEM((BPW, D), jnp.float32),
          pltpu.VMEM((BPW, D), jnp.float32),
          pltpu.SemaphoreType.DMA,
      ],
      compiler_params=pltpu.CompilerParams(use_tc_tiling_on_sc=False),
  )
  def gather_kernel(uids, mids, utab, mtab, out_u, out_m,
                    idx_u, idx_m, rows_u, rows_m, sem):
    wid = lax.axis_index("s") * NC + lax.axis_index("c")
    pltpu.sync_copy(uids.at[wid], idx_u)
    pltpu.sync_copy(mids.at[wid], idx_m)
    copies = []
    for j in range(NCH):
      copies.append(pltpu.async_copy(
          utab.at[idx_u.at[j]], rows_u.at[pl.ds(j * CHUNK, CHUNK)], sem))
      copies.append(pltpu.async_copy(
          mtab.at[idx_m.at[j]], rows_m.at[pl.ds(j * CHUNK, CHUNK)], sem))
    for c in copies:
      c.wait()
    pltpu.sync_copy(rows_u, out_u.at[wid])
    pltpu.sync_copy(rows_m, out_m.at[wid])

  return gather_kernel


_gather = _build_gather()


def _mlp_body(u_ref, m_ref, w1u_ref, w1m_ref, b1_ref, w2_ref, b2_ref,
              w3r_ref, b3_ref, o_ref):
  h = jnp.dot(u_ref[...], w1u_ref[...], preferred_element_type=jnp.float32)
  h = h + jnp.dot(m_ref[...], w1m_ref[...], preferred_element_type=jnp.float32)
  h = jnp.maximum(h + b1_ref[...], 0.0)
  h = jnp.maximum(
      jnp.dot(h, w2_ref[...], preferred_element_type=jnp.float32)
      + b2_ref[...], 0.0)
  o_ref[...] = jnp.sum(h * w3r_ref[...], axis=1) + b3_ref[0, 0]


_mlp = pl.pallas_call(
    _mlp_body,
    grid=(GRID,),
    in_specs=[
        pl.BlockSpec((BLK, D), lambda i: (i, 0)),
        pl.BlockSpec((BLK, D), lambda i: (i, 0)),
        pl.BlockSpec((D, H1), lambda i: (0, 0)),
        pl.BlockSpec((D, H1), lambda i: (0, 0)),
        pl.BlockSpec((1, H1), lambda i: (0, 0)),
        pl.BlockSpec((H1, H2), lambda i: (0, 0)),
        pl.BlockSpec((1, H2), lambda i: (0, 0)),
        pl.BlockSpec((1, H2), lambda i: (0, 0)),
        pl.BlockSpec((1, 1), lambda i: (0, 0)),
    ],
    out_specs=pl.BlockSpec((BLK,), lambda i: (i,)),
    out_shape=jax.ShapeDtypeStruct((B,), jnp.float32),
)


@jax.jit
def kernel(user_ids, movie_ids, user_table, movie_table, W1, b1, W2, b2,
           W3, b3):
  uids = user_ids.astype(jnp.int32).reshape(NW, NCH, CHUNK)
  mids = movie_ids.astype(jnp.int32).reshape(NW, NCH, CHUNK)
  rows_u, rows_m = _gather(uids, mids, user_table, movie_table)
  u = rows_u.reshape(B, D)
  m = rows_m.reshape(B, D)
  return _mlp(u, m,
              W1[:D], W1[D:], b1.reshape(1, H1),
              W2, b2.reshape(1, H2),
              W3.reshape(1, H2), b3.reshape(1, 1))


# speedup vs baseline: 1.0745x; 1.0745x over previous
import functools

import jax
import jax.numpy as jnp
from jax import lax
from jax.experimental import pallas as pl
from jax.experimental.pallas import tpu as pltpu
from jax.experimental.pallas import tpu_sc as plsc

V = 1_000_000
B = 16384
D = 32
H1 = 64
H2 = 32

NC = 2
NS = 16
NW = NC * NS
L = 16
B2 = 2 * B
BPW = B2 // NW

TBLK = 8192
TGRID = pl.cdiv(V, TBLK)

BLK = 2048
GRID = B // BLK


def _tr_body(t_ref, o_ref):
  xT = t_ref[...].T
  x3 = xT.reshape(TBLK // 4, 4, D)
  o_ref[...] = jnp.concatenate([x3[:, c, :] for c in range(4)], axis=1)


_tr = pl.pallas_call(
    _tr_body,
    grid=(TGRID,),
    in_specs=[pl.BlockSpec((D, TBLK), lambda i: (0, i))],
    out_specs=pl.BlockSpec((TBLK * D // 128, 128), lambda i: (i, 0)),
    out_shape=jax.ShapeDtypeStruct((V * D // 128, 128), jnp.float32),
)


def _build_gather():
  mesh = plsc.VectorSubcoreMesh(core_axis_name="c", subcore_axis_name="s")

  @functools.partial(
      pl.kernel,
      mesh=mesh,
      out_type=[
          jax.ShapeDtypeStruct((B2, L), jnp.float32),
          jax.ShapeDtypeStruct((B2, L), jnp.float32),
      ],
      scratch_types=[
          pltpu.VMEM((BPW,), jnp.int32),
          pltpu.VMEM((BPW,), jnp.int32),
          pltpu.VMEM((BPW, L), jnp.float32),
          pltpu.VMEM((BPW, L), jnp.float32),
          pltpu.SemaphoreType.DMA,
      ],
      compiler_params=pltpu.CompilerParams(use_tc_tiling_on_sc=False),
  )
  def gather_kernel(uids, mids, utab, mtab, out_u, out_m,
                    idx_u, idx_m, ru, rm, sem):
    wid = lax.axis_index("s") * NC + lax.axis_index("c")
    base = wid * BPW
    pltpu.sync_copy(uids.at[pl.ds(base, BPW)], idx_u)
    pltpu.sync_copy(mids.at[pl.ds(base, BPW)], idx_m)
    cu = pltpu.async_copy(utab.at[idx_u], ru, sem)
    cm = pltpu.async_copy(mtab.at[idx_m], rm, sem)
    cu.wait()
    cm.wait()
    pltpu.sync_copy(ru, out_u.at[pl.ds(base, BPW)])
    pltpu.sync_copy(rm, out_m.at[pl.ds(base, BPW)])

  return gather_kernel


_gather = _build_gather()


def _mlp_body(u_ref, m_ref, w1u_ref, w1m_ref, b1_ref, w2_ref, b2_ref,
              w3_ref, b3_ref, o_ref):
  h = jnp.dot(u_ref[...], w1u_ref[...], preferred_element_type=jnp.float32)
  h = h + jnp.dot(m_ref[...], w1m_ref[...], preferred_element_type=jnp.float32)
  h = jnp.maximum(h + b1_ref[...], 0.0)
  h = jnp.maximum(
      jnp.dot(h, w2_ref[...], preferred_element_type=jnp.float32)
      + b2_ref[...], 0.0)
  o_ref[...] = jnp.sum(h * w3_ref[...], axis=1) + b3_ref[0, 0]


_mlp = pl.pallas_call(
    _mlp_body,
    grid=(GRID,),
    in_specs=[
        pl.BlockSpec((BLK, D), lambda i: (i, 0)),
        pl.BlockSpec((BLK, D), lambda i: (i, 0)),
        pl.BlockSpec((D, H1), lambda i: (0, 0)),
        pl.BlockSpec((D, H1), lambda i: (0, 0)),
        pl.BlockSpec((1, H1), lambda i: (0, 0)),
        pl.BlockSpec((H1, H2), lambda i: (0, 0)),
        pl.BlockSpec((1, H2), lambda i: (0, 0)),
        pl.BlockSpec((1, H2), lambda i: (0, 0)),
        pl.BlockSpec((1, 1), lambda i: (0, 0)),
    ],
    out_specs=pl.BlockSpec((BLK,), lambda i: (i,)),
    out_shape=jax.ShapeDtypeStruct((B,), jnp.float32),
)


@jax.jit
def kernel(user_ids, movie_ids, user_table, movie_table, W1, b1, W2, b2,
           W3, b3):
  two = jnp.int32(2)
  uids = (user_ids.astype(jnp.int32)[:, None] * two
          + jnp.arange(2, dtype=jnp.int32)).reshape(B2)
  mids = (movie_ids.astype(jnp.int32)[:, None] * two
          + jnp.arange(2, dtype=jnp.int32)).reshape(B2)
  ulin = _tr(user_table.T).reshape(2 * V, L)
  mlin = _tr(movie_table.T).reshape(2 * V, L)
  u_half, m_half = _gather(uids, mids, ulin, mlin)
  u_rows = u_half.reshape(B, D)
  m_rows = m_half.reshape(B, D)
  return _mlp(u_rows, m_rows,
              W1[:D], W1[D:], b1.reshape(1, H1),
              W2, b2.reshape(1, H2),
              W3.reshape(1, H2), b3.reshape(1, 1))
